# baseline (speedup 1.0000x reference)
import functools

import jax
import jax.numpy as jnp
from jax import lax
from jax.experimental import pallas as pl
from jax.experimental.pallas import tpu as pltpu
from jax.experimental.pallas import tpu_sc as plsc

N = 10000
D_IN = 128
D_HID = 128
D_OUT = 47
D_OUT_PAD = 128

NC = 2
NS = 16
NW = NC * NS
CH = 128
CPW0 = 76
CPW1 = 82
ACC_R = 10240
TILE_R = ACC_R // NS
DUMMY = N + 16

BR = 1000


def _cdiv(a, b):
    return (a + b - 1) // b


def _make_sc_agg(D, with_deg):
    assert CPW0 % 2 == 0 and CPW1 % 2 == 0
    mesh = plsc.VectorSubcoreMesh(core_axis_name="c", subcore_axis_name="s")
    agg_t = jax.ShapeDtypeStruct((NC, ACC_R, D), jnp.float32)
    out_type = (agg_t, jax.ShapeDtypeStruct((NC, ACC_R), jnp.float32)) if with_deg else agg_t
    scratch = [
        pltpu.VMEM((CH,), jnp.int32),
        pltpu.VMEM((CH,), jnp.int32),
        pltpu.VMEM((CH,), jnp.int32),
        pltpu.VMEM((CH,), jnp.int32),
        pltpu.VMEM((CH, D), jnp.float32),
        pltpu.VMEM((CH, D), jnp.float32),
        pltpu.VMEM_SHARED((ACC_R, D), jnp.float32),
        pltpu.SemaphoreType.DMA,
        pltpu.SemaphoreType.DMA,
        pltpu.SemaphoreType.DMA,
        pltpu.SemaphoreType.DMA,
        pltpu.SemaphoreType.DMA,
        pltpu.SemaphoreType.DMA,
    ]
    if with_deg:
        scratch += [
            pltpu.VMEM((CH,), jnp.float32),
            pltpu.VMEM_SHARED((ACC_R,), jnp.float32),
        ]

    def body(hw, srcm, dstm, zrows, zdeg, ones, *rest):
        if with_deg:
            (agg_out, deg_out, sa, sb, da, db, rows_a, rows_b, acc,
             gsem_a, gsem_b, ssem_a, ssem_b, dsem_a, dsem_b, ones_v, dacc) = rest
        else:
            (agg_out, sa, sb, da, db, rows_a, rows_b, acc,
             gsem_a, gsem_b, ssem_a, ssem_b, dsem_a, dsem_b) = rest
        cid = lax.axis_index("c")
        sid = lax.axis_index("s")
        base = sid * TILE_R
        c0 = jnp.where(cid == 0, sid * CPW0, NS * CPW0 + sid * CPW1)
        cpw = jnp.where(cid == 0, CPW0, CPW1)

        pltpu.sync_copy(zrows, acc.at[pl.ds(base, TILE_R)])
        if with_deg:
            pltpu.sync_copy(zdeg, dacc.at[pl.ds(base, TILE_R)])
            pltpu.sync_copy(ones, ones_v)

        plsc.subcore_barrier()

        pltpu.async_copy(srcm.at[c0], sa, ssem_a)
        pltpu.async_copy(dstm.at[c0], da, dsem_a)
        pltpu.async_copy(srcm.at[c0 + 1], sb, ssem_b)
        pltpu.async_copy(dstm.at[c0 + 1], db, dsem_b)
        pltpu.make_async_copy(srcm.at[c0], sa, ssem_a).wait()
        pltpu.async_copy(hw.at[sa], rows_a, gsem_a)

        def step(j, _):
            i = j * 2
            nxt_a = jnp.where(i + 2 < cpw, i + 2, 0)
            nxt_b = jnp.where(i + 3 < cpw, i + 3, 1)
            pltpu.make_async_copy(srcm.at[c0 + i + 1], sb, ssem_b).wait()
            pltpu.async_copy(hw.at[sb], rows_b, gsem_b)
            pltpu.make_async_copy(hw.at[sa], rows_a, gsem_a).wait()
            pltpu.async_copy(srcm.at[c0 + nxt_a], sa, ssem_a)
            pltpu.make_async_copy(dstm.at[c0 + i], da, dsem_a).wait()
            pltpu.sync_copy(rows_a, acc.at[da], add=True)
            if with_deg:
                pltpu.sync_copy(ones_v, dacc.at[da], add=True)
            pltpu.async_copy(dstm.at[c0 + nxt_a], da, dsem_a)
            pltpu.make_async_copy(srcm.at[c0 + nxt_a], sa, ssem_a).wait()
            pltpu.async_copy(hw.at[sa], rows_a, gsem_a)
            pltpu.make_async_copy(hw.at[sb], rows_b, gsem_b).wait()
            pltpu.async_copy(srcm.at[c0 + nxt_b], sb, ssem_b)
            pltpu.make_async_copy(dstm.at[c0 + i + 1], db, dsem_b).wait()
            pltpu.sync_copy(rows_b, acc.at[db], add=True)
            if with_deg:
                pltpu.sync_copy(ones_v, dacc.at[db], add=True)
            pltpu.async_copy(dstm.at[c0 + nxt_b], db, dsem_b)
            return 0
        lax.fori_loop(0, cpw // 2, step, 0)

        pltpu.make_async_copy(hw.at[sa], rows_a, gsem_a).wait()
        pltpu.make_async_copy(srcm.at[c0], sb, ssem_b).wait()
        pltpu.make_async_copy(dstm.at[c0], da, dsem_a).wait()
        pltpu.make_async_copy(dstm.at[c0], db, dsem_b).wait()

        plsc.subcore_barrier()

        pltpu.sync_copy(acc.at[pl.ds(base, TILE_R)],
                        agg_out.at[cid, pl.ds(base, TILE_R)])
        if with_deg:
            pltpu.sync_copy(dacc.at[pl.ds(base, TILE_R)],
                            deg_out.at[cid, pl.ds(base, TILE_R)])

    return pl.kernel(body, mesh=mesh, out_type=out_type,
                     scratch_types=scratch)


def _mm_first(x, Ws, Wn, b, Dout):
    def body(x_ref, ws_ref, wn_ref, b_ref, hs_ref, hw_ref):
        h = x_ref[...]
        hs_ref[...] = jnp.dot(h, ws_ref[...],
                              preferred_element_type=jnp.float32) + b_ref[...]
        hw_ref[...] = jnp.dot(h, wn_ref[...],
                              preferred_element_type=jnp.float32)
    Din = x.shape[1]
    return pl.pallas_call(
        body,
        grid=(N // BR,),
        in_specs=[
            pl.BlockSpec((BR, Din), lambda i: (i, 0)),
            pl.BlockSpec((Din, Dout), lambda i: (0, 0)),
            pl.BlockSpec((Din, Dout), lambda i: (0, 0)),
            pl.BlockSpec((1, Dout), lambda i: (0, 0)),
        ],
        out_specs=[
            pl.BlockSpec((BR, Dout), lambda i: (i, 0)),
            pl.BlockSpec((BR, Dout), lambda i: (i, 0)),
        ],
        out_shape=[
            jax.ShapeDtypeStruct((N, Dout), jnp.float32),
            jax.ShapeDtypeStruct((N, Dout), jnp.float32),
        ],
    )(x, Ws, Wn, b)


def _mm_mid(hs, aggp, degp3, Ws, Wn, b, Dout):
    def body(hs_ref, agg_ref, deg_ref, ws_ref, wn_ref, b_ref, hso_ref, hwo_ref):
        inv = 1.0 / jnp.maximum(deg_ref[0] + deg_ref[1], 1.0)
        h = jnp.maximum(hs_ref[...] + (agg_ref[0] + agg_ref[1]) * inv, 0.0)
        hso_ref[...] = jnp.dot(h, ws_ref[...],
                               preferred_element_type=jnp.float32) + b_ref[...]
        hwo_ref[...] = jnp.dot(h, wn_ref[...],
                               preferred_element_type=jnp.float32)
    Din = hs.shape[1]
    return pl.pallas_call(
        body,
        grid=(N // BR,),
        in_specs=[
            pl.BlockSpec((BR, Din), lambda i: (i, 0)),
            pl.BlockSpec((NC, BR, Din), lambda i: (0, i, 0)),
            pl.BlockSpec((NC, BR, 1), lambda i: (0, i, 0)),
            pl.BlockSpec((Din, Dout), lambda i: (0, 0)),
            pl.BlockSpec((Din, Dout), lambda i: (0, 0)),
            pl.BlockSpec((1, Dout), lambda i: (0, 0)),
        ],
        out_specs=[
            pl.BlockSpec((BR, Dout), lambda i: (i, 0)),
            pl.BlockSpec((BR, Dout), lambda i: (i, 0)),
        ],
        out_shape=[
            jax.ShapeDtypeStruct((N, Dout), jnp.float32),
            jax.ShapeDtypeStruct((N, Dout), jnp.float32),
        ],
    )(hs, aggp, degp3, Ws, Wn, b)


def _mm_last(hs, aggp, degp3):
    D = hs.shape[1]
    def body(hs_ref, agg_ref, deg_ref, o_ref):
        inv = 1.0 / jnp.maximum(deg_ref[0] + deg_ref[1], 1.0)
        o_ref[...] = hs_ref[...] + (agg_ref[0] + agg_ref[1]) * inv
    return pl.pallas_call(
        body,
        grid=(N // BR,),
        in_specs=[
            pl.BlockSpec((BR, D), lambda i: (i, 0)),
            pl.BlockSpec((NC, BR, D), lambda i: (0, i, 0)),
            pl.BlockSpec((NC, BR, 1), lambda i: (0, i, 0)),
        ],
        out_specs=pl.BlockSpec((BR, D), lambda i: (i, 0)),
        out_shape=jax.ShapeDtypeStruct((N, D), jnp.float32),
    )(hs, aggp, degp3)


def kernel(x, edge_index, W_self0, W_neigh0, b0, W_self1, W_neigh1, b1,
           W_self2, W_neigh2, b2):
    src = edge_index[0].astype(jnp.int32)
    dst = edge_index[1].astype(jnp.int32)
    E = src.shape[0]
    nchunk = NS * (CPW0 + CPW1)
    ep = nchunk * CH
    assert ep >= E
    pad_src = jnp.arange(ep - E, dtype=jnp.int32) % N
    pad_dst = N + jnp.arange(ep - E, dtype=jnp.int32) % (ACC_R - N)
    srcm = jnp.concatenate([src, pad_src]).reshape(nchunk, CH)
    dstm = jnp.concatenate([dst, pad_dst]).reshape(nchunk, CH)
    zrows = jnp.zeros((TILE_R, D_HID), jnp.float32)
    zdeg = jnp.zeros((TILE_R,), jnp.float32)
    ones = jnp.ones((CH,), jnp.float32)

    sc_agg_deg = _make_sc_agg(D_HID, True)
    sc_agg = _make_sc_agg(D_HID, False)

    b0r = b0.reshape(1, D_HID)
    b1r = b1.reshape(1, D_HID)
    Ws2p = jnp.pad(W_self2, ((0, 0), (0, D_OUT_PAD - D_OUT)))
    Wn2p = jnp.pad(W_neigh2, ((0, 0), (0, D_OUT_PAD - D_OUT)))
    b2r = jnp.pad(b2, (0, D_OUT_PAD - D_OUT)).reshape(1, D_OUT_PAD)

    hs0, hw0 = _mm_first(x, W_self0, W_neigh0, b0r, D_HID)
    aggp0, degp = sc_agg_deg(hw0, srcm, dstm, zrows, zdeg, ones)
    degp3 = degp.reshape(NC, ACC_R, 1)
    hs1, hw1 = _mm_mid(hs0, aggp0, degp3, W_self1, W_neigh1, b1r, D_HID)
    aggp1 = sc_agg(hw1, srcm, dstm, zrows, zdeg, ones)
    hs2, hw2 = _mm_mid(hs1, aggp1, degp3, Ws2p, Wn2p, b2r, D_OUT_PAD)
    aggp2 = sc_agg(hw2, srcm, dstm, zrows, zdeg, ones)
    out = _mm_last(hs2, aggp2, degp3)
    return out[:, :D_OUT]


# speedup vs baseline: 1.0138x; 1.0138x over previous
import functools

import jax
import jax.numpy as jnp
from jax import lax
from jax.experimental import pallas as pl
from jax.experimental.pallas import tpu as pltpu
from jax.experimental.pallas import tpu_sc as plsc

N = 10000
D_IN = 128
D_HID = 128
D_OUT = 47
D_OUT_PAD = 128

NC = 2
NS = 16
NW = NC * NS
CH = 128
CPW0 = 78
CPW1 = 80
ACC_R = 10240
TILE_R = ACC_R // NS
DUMMY = N + 16

BR = 1000


def _cdiv(a, b):
    return (a + b - 1) // b


def _make_sc_agg(D, with_deg):
    assert CPW0 % 2 == 0 and CPW1 % 2 == 0
    mesh = plsc.VectorSubcoreMesh(core_axis_name="c", subcore_axis_name="s")
    agg_t = jax.ShapeDtypeStruct((NC, ACC_R, D), jnp.float32)
    out_type = (agg_t, jax.ShapeDtypeStruct((NC, ACC_R), jnp.float32)) if with_deg else agg_t
    scratch = [
        pltpu.VMEM((CH,), jnp.int32),
        pltpu.VMEM((CH,), jnp.int32),
        pltpu.VMEM((CH,), jnp.int32),
        pltpu.VMEM((CH,), jnp.int32),
        pltpu.VMEM((CH, D), jnp.float32),
        pltpu.VMEM((CH, D), jnp.float32),
        pltpu.VMEM_SHARED((ACC_R, D), jnp.float32),
        pltpu.SemaphoreType.DMA,
        pltpu.SemaphoreType.DMA,
        pltpu.SemaphoreType.DMA,
        pltpu.SemaphoreType.DMA,
        pltpu.SemaphoreType.DMA,
        pltpu.SemaphoreType.DMA,
    ]
    if with_deg:
        scratch += [
            pltpu.VMEM((CH,), jnp.float32),
            pltpu.VMEM_SHARED((ACC_R,), jnp.float32),
        ]

    def body(hw, srcm, dstm, zrows, zdeg, ones, *rest):
        if with_deg:
            (agg_out, deg_out, sa, sb, da, db, rows_a, rows_b, acc,
             gsem_a, gsem_b, ssem_a, ssem_b, dsem_a, dsem_b, ones_v, dacc) = rest
        else:
            (agg_out, sa, sb, da, db, rows_a, rows_b, acc,
             gsem_a, gsem_b, ssem_a, ssem_b, dsem_a, dsem_b) = rest
        cid = lax.axis_index("c")
        sid = lax.axis_index("s")
        base = sid * TILE_R
        c0 = jnp.where(cid == 0, sid * CPW0, NS * CPW0 + sid * CPW1)
        cpw = jnp.where(cid == 0, CPW0, CPW1)

        pltpu.sync_copy(zrows, acc.at[pl.ds(base, TILE_R)])
        if with_deg:
            pltpu.sync_copy(zdeg, dacc.at[pl.ds(base, TILE_R)])
            pltpu.sync_copy(ones, ones_v)

        plsc.subcore_barrier()

        pltpu.async_copy(srcm.at[c0], sa, ssem_a)
        pltpu.async_copy(dstm.at[c0], da, dsem_a)
        pltpu.async_copy(srcm.at[c0 + 1], sb, ssem_b)
        pltpu.async_copy(dstm.at[c0 + 1], db, dsem_b)
        pltpu.make_async_copy(srcm.at[c0], sa, ssem_a).wait()
        pltpu.async_copy(hw.at[sa], rows_a, gsem_a)

        def step(j, _):
            i = j * 2
            nxt_a = jnp.where(i + 2 < cpw, i + 2, 0)
            nxt_b = jnp.where(i + 3 < cpw, i + 3, 1)
            pltpu.make_async_copy(srcm.at[c0 + i + 1], sb, ssem_b).wait()
            pltpu.async_copy(hw.at[sb], rows_b, gsem_b)
            pltpu.make_async_copy(hw.at[sa], rows_a, gsem_a).wait()
            pltpu.async_copy(srcm.at[c0 + nxt_a], sa, ssem_a)
            pltpu.make_async_copy(dstm.at[c0 + i], da, dsem_a).wait()
            pltpu.sync_copy(rows_a, acc.at[da], add=True)
            if with_deg:
                pltpu.sync_copy(ones_v, dacc.at[da], add=True)
            pltpu.async_copy(dstm.at[c0 + nxt_a], da, dsem_a)
            pltpu.make_async_copy(srcm.at[c0 + nxt_a], sa, ssem_a).wait()
            pltpu.async_copy(hw.at[sa], rows_a, gsem_a)
            pltpu.make_async_copy(hw.at[sb], rows_b, gsem_b).wait()
            pltpu.async_copy(srcm.at[c0 + nxt_b], sb, ssem_b)
            pltpu.make_async_copy(dstm.at[c0 + i + 1], db, dsem_b).wait()
            pltpu.sync_copy(rows_b, acc.at[db], add=True)
            if with_deg:
                pltpu.sync_copy(ones_v, dacc.at[db], add=True)
            pltpu.async_copy(dstm.at[c0 + nxt_b], db, dsem_b)
            return 0
        lax.fori_loop(0, cpw // 2, step, 0)

        pltpu.make_async_copy(hw.at[sa], rows_a, gsem_a).wait()
        pltpu.make_async_copy(srcm.at[c0], sb, ssem_b).wait()
        pltpu.make_async_copy(dstm.at[c0], da, dsem_a).wait()
        pltpu.make_async_copy(dstm.at[c0], db, dsem_b).wait()

        plsc.subcore_barrier()

        pltpu.sync_copy(acc.at[pl.ds(base, TILE_R)],
                        agg_out.at[cid, pl.ds(base, TILE_R)])
        if with_deg:
            pltpu.sync_copy(dacc.at[pl.ds(base, TILE_R)],
                            deg_out.at[cid, pl.ds(base, TILE_R)])

    return pl.kernel(body, mesh=mesh, out_type=out_type,
                     scratch_types=scratch)


def _mm_first(x, Ws, Wn, b, Dout):
    def body(x_ref, ws_ref, wn_ref, b_ref, hs_ref, hw_ref):
        h = x_ref[...]
        hs_ref[...] = jnp.dot(h, ws_ref[...],
                              preferred_element_type=jnp.float32) + b_ref[...]
        hw_ref[...] = jnp.dot(h, wn_ref[...],
                              preferred_element_type=jnp.float32)
    Din = x.shape[1]
    return pl.pallas_call(
        body,
        grid=(N // BR,),
        in_specs=[
            pl.BlockSpec((BR, Din), lambda i: (i, 0)),
            pl.BlockSpec((Din, Dout), lambda i: (0, 0)),
            pl.BlockSpec((Din, Dout), lambda i: (0, 0)),
            pl.BlockSpec((1, Dout), lambda i: (0, 0)),
        ],
        out_specs=[
            pl.BlockSpec((BR, Dout), lambda i: (i, 0)),
            pl.BlockSpec((BR, Dout), lambda i: (i, 0)),
        ],
        out_shape=[
            jax.ShapeDtypeStruct((N, Dout), jnp.float32),
            jax.ShapeDtypeStruct((N, Dout), jnp.float32),
        ],
    )(x, Ws, Wn, b)


def _mm_mid(hs, aggp, degp3, Ws, Wn, b, Dout):
    def body(hs_ref, agg_ref, deg_ref, ws_ref, wn_ref, b_ref, hso_ref, hwo_ref):
        inv = 1.0 / jnp.maximum(deg_ref[0] + deg_ref[1], 1.0)
        h = jnp.maximum(hs_ref[...] + (agg_ref[0] + agg_ref[1]) * inv, 0.0)
        hso_ref[...] = jnp.dot(h, ws_ref[...],
                               preferred_element_type=jnp.float32) + b_ref[...]
        hwo_ref[...] = jnp.dot(h, wn_ref[...],
                               preferred_element_type=jnp.float32)
    Din = hs.shape[1]
    return pl.pallas_call(
        body,
        grid=(N // BR,),
        in_specs=[
            pl.BlockSpec((BR, Din), lambda i: (i, 0)),
            pl.BlockSpec((NC, BR, Din), lambda i: (0, i, 0)),
            pl.BlockSpec((NC, BR, 1), lambda i: (0, i, 0)),
            pl.BlockSpec((Din, Dout), lambda i: (0, 0)),
            pl.BlockSpec((Din, Dout), lambda i: (0, 0)),
            pl.BlockSpec((1, Dout), lambda i: (0, 0)),
        ],
        out_specs=[
            pl.BlockSpec((BR, Dout), lambda i: (i, 0)),
            pl.BlockSpec((BR, Dout), lambda i: (i, 0)),
        ],
        out_shape=[
            jax.ShapeDtypeStruct((N, Dout), jnp.float32),
            jax.ShapeDtypeStruct((N, Dout), jnp.float32),
        ],
    )(hs, aggp, degp3, Ws, Wn, b)


def _mm_last(hs, aggp, degp3):
    D = hs.shape[1]
    def body(hs_ref, agg_ref, deg_ref, o_ref):
        inv = 1.0 / jnp.maximum(deg_ref[0] + deg_ref[1], 1.0)
        o_ref[...] = hs_ref[...] + (agg_ref[0] + agg_ref[1]) * inv
    return pl.pallas_call(
        body,
        grid=(N // BR,),
        in_specs=[
            pl.BlockSpec((BR, D), lambda i: (i, 0)),
            pl.BlockSpec((NC, BR, D), lambda i: (0, i, 0)),
            pl.BlockSpec((NC, BR, 1), lambda i: (0, i, 0)),
        ],
        out_specs=pl.BlockSpec((BR, D), lambda i: (i, 0)),
        out_shape=jax.ShapeDtypeStruct((N, D), jnp.float32),
    )(hs, aggp, degp3)


def kernel(x, edge_index, W_self0, W_neigh0, b0, W_self1, W_neigh1, b1,
           W_self2, W_neigh2, b2):
    src = edge_index[0].astype(jnp.int32)
    dst = edge_index[1].astype(jnp.int32)
    E = src.shape[0]
    nchunk = NS * (CPW0 + CPW1)
    ep = nchunk * CH
    assert ep >= E
    pad_src = jnp.arange(ep - E, dtype=jnp.int32) % N
    pad_dst = N + jnp.arange(ep - E, dtype=jnp.int32) % (ACC_R - N)
    srcm = jnp.concatenate([src, pad_src]).reshape(nchunk, CH)
    dstm = jnp.concatenate([dst, pad_dst]).reshape(nchunk, CH)
    zrows = jnp.zeros((TILE_R, D_HID), jnp.float32)
    zdeg = jnp.zeros((TILE_R,), jnp.float32)
    ones = jnp.ones((CH,), jnp.float32)

    sc_agg_deg = _make_sc_agg(D_HID, True)
    sc_agg = _make_sc_agg(D_HID, False)

    b0r = b0.reshape(1, D_HID)
    b1r = b1.reshape(1, D_HID)
    Ws2p = jnp.pad(W_self2, ((0, 0), (0, D_OUT_PAD - D_OUT)))
    Wn2p = jnp.pad(W_neigh2, ((0, 0), (0, D_OUT_PAD - D_OUT)))
    b2r = jnp.pad(b2, (0, D_OUT_PAD - D_OUT)).reshape(1, D_OUT_PAD)

    hs0, hw0 = _mm_first(x, W_self0, W_neigh0, b0r, D_HID)
    aggp0, degp = sc_agg_deg(hw0, srcm, dstm, zrows, zdeg, ones)
    degp3 = degp.reshape(NC, ACC_R, 1)
    hs1, hw1 = _mm_mid(hs0, aggp0, degp3, W_self1, W_neigh1, b1r, D_HID)
    aggp1 = sc_agg(hw1, srcm, dstm, zrows, zdeg, ones)
    hs2, hw2 = _mm_mid(hs1, aggp1, degp3, Ws2p, Wn2p, b2r, D_OUT_PAD)
    aggp2 = sc_agg(hw2, srcm, dstm, zrows, zdeg, ones)
    out = _mm_last(hs2, aggp2, degp3)
    return out[:, :D_OUT]


# speedup vs baseline: 1.0139x; 1.0000x over previous
import jax
import jax.numpy as jnp
from jax import lax
from jax.experimental import pallas as pl
from jax.experimental.pallas import tpu as pltpu
from jax.experimental.pallas import tpu_sc as plsc

N = 10000
D_IN = 128
D_HID = 128
D_OUT = 47
D_OUT_PAD = 128

NC = 2
NS = 16
NW = NC * NS
CH = 128
CPW0 = 78
CPW1 = 80
ACC_R = 10240
TILE_R = ACC_R // NS

BR = 1000


def _cdiv(a, b):
    return (a + b - 1) // b


def _make_sc_agg(D, with_deg):
    assert CPW0 % 2 == 0 and CPW1 % 2 == 0
    mesh = plsc.VectorSubcoreMesh(core_axis_name="c", subcore_axis_name="s")
    agg_t = jax.ShapeDtypeStruct((NC, ACC_R, D), jnp.float32)
    out_type = (agg_t, jax.ShapeDtypeStruct((NC, ACC_R), jnp.float32)) if with_deg else agg_t
    scratch = [
        pltpu.VMEM((CH,), jnp.int32),
        pltpu.VMEM((CH,), jnp.int32),
        pltpu.VMEM((CH,), jnp.int32),
        pltpu.VMEM((CH,), jnp.int32),
        pltpu.VMEM((CH, D), jnp.float32),
        pltpu.VMEM((CH, D), jnp.float32),
        pltpu.VMEM_SHARED((ACC_R, D), jnp.float32),
        pltpu.SemaphoreType.DMA,
        pltpu.SemaphoreType.DMA,
        pltpu.SemaphoreType.DMA,
        pltpu.SemaphoreType.DMA,
        pltpu.SemaphoreType.DMA,
        pltpu.SemaphoreType.DMA,
    ]
    if with_deg:
        scratch += [
            pltpu.VMEM((CH,), jnp.float32),
            pltpu.VMEM_SHARED((ACC_R,), jnp.float32),
        ]

    def body(hw, srcm, dstm, zrows, zdeg, ones, *rest):
        if with_deg:
            (agg_out, deg_out, sa, sb, da, db, rows_a, rows_b, acc,
             gsem_a, gsem_b, ssem_a, ssem_b, dsem_a, dsem_b, ones_v, dacc) = rest
        else:
            (agg_out, sa, sb, da, db, rows_a, rows_b, acc,
             gsem_a, gsem_b, ssem_a, ssem_b, dsem_a, dsem_b) = rest
        cid = lax.axis_index("c")
        sid = lax.axis_index("s")
        base = sid * TILE_R
        c0 = jnp.where(cid == 0, sid * CPW0, NS * CPW0 + sid * CPW1)
        cpw = jnp.where(cid == 0, CPW0, CPW1)

        pltpu.sync_copy(zrows, acc.at[pl.ds(base, TILE_R)])
        if with_deg:
            pltpu.sync_copy(zdeg, dacc.at[pl.ds(base, TILE_R)])
            pltpu.sync_copy(ones, ones_v)

        plsc.subcore_barrier()

        pltpu.async_copy(srcm.at[c0], sa, ssem_a)
        pltpu.async_copy(dstm.at[c0], da, dsem_a)
        pltpu.async_copy(srcm.at[c0 + 1], sb, ssem_b)
        pltpu.async_copy(dstm.at[c0 + 1], db, dsem_b)
        pltpu.make_async_copy(srcm.at[c0], sa, ssem_a).wait()
        pltpu.async_copy(hw.at[sa], rows_a, gsem_a)

        def step(j, _):
            i = j * 2
            nxt_a = jnp.where(i + 2 < cpw, i + 2, 0)
            nxt_b = jnp.where(i + 3 < cpw, i + 3, 1)
            pltpu.make_async_copy(srcm.at[c0 + i + 1], sb, ssem_b).wait()
            pltpu.async_copy(hw.at[sb], rows_b, gsem_b)
            pltpu.make_async_copy(hw.at[sa], rows_a, gsem_a).wait()
            pltpu.async_copy(srcm.at[c0 + nxt_a], sa, ssem_a)
            pltpu.make_async_copy(dstm.at[c0 + i], da, dsem_a).wait()
            pltpu.sync_copy(rows_a, acc.at[da], add=True)
            if with_deg:
                pltpu.sync_copy(ones_v, dacc.at[da], add=True)
            pltpu.async_copy(dstm.at[c0 + nxt_a], da, dsem_a)
            pltpu.make_async_copy(srcm.at[c0 + nxt_a], sa, ssem_a).wait()
            pltpu.async_copy(hw.at[sa], rows_a, gsem_a)
            pltpu.make_async_copy(hw.at[sb], rows_b, gsem_b).wait()
            pltpu.async_copy(srcm.at[c0 + nxt_b], sb, ssem_b)
            pltpu.make_async_copy(dstm.at[c0 + i + 1], db, dsem_b).wait()
            pltpu.sync_copy(rows_b, acc.at[db], add=True)
            if with_deg:
                pltpu.sync_copy(ones_v, dacc.at[db], add=True)
            pltpu.async_copy(dstm.at[c0 + nxt_b], db, dsem_b)
            return 0
        lax.fori_loop(0, cpw // 2, step, 0)

        pltpu.make_async_copy(hw.at[sa], rows_a, gsem_a).wait()
        pltpu.make_async_copy(srcm.at[c0], sb, ssem_b).wait()
        pltpu.make_async_copy(dstm.at[c0], da, dsem_a).wait()
        pltpu.make_async_copy(dstm.at[c0], db, dsem_b).wait()

        plsc.subcore_barrier()

        pltpu.sync_copy(acc.at[pl.ds(base, TILE_R)],
                        agg_out.at[cid, pl.ds(base, TILE_R)])
        if with_deg:
            pltpu.sync_copy(dacc.at[pl.ds(base, TILE_R)],
                            deg_out.at[cid, pl.ds(base, TILE_R)])

    return pl.kernel(body, mesh=mesh, out_type=out_type,
                     scratch_types=scratch)


def _mm_first(x, Ws, Wn, b, Dout):
    def body(x_ref, ws_ref, wn_ref, b_ref, hs_ref, hw_ref):
        h = x_ref[...]
        hs_ref[...] = jnp.dot(h, ws_ref[...],
                              preferred_element_type=jnp.float32) + b_ref[...]
        hw_ref[...] = jnp.dot(h, wn_ref[...],
                              preferred_element_type=jnp.float32)
    Din = x.shape[1]
    return pl.pallas_call(
        body,
        grid=(N // BR,),
        in_specs=[
            pl.BlockSpec((BR, Din), lambda i: (i, 0)),
            pl.BlockSpec((Din, Dout), lambda i: (0, 0)),
            pl.BlockSpec((Din, Dout), lambda i: (0, 0)),
            pl.BlockSpec((1, Dout), lambda i: (0, 0)),
        ],
        out_specs=[
            pl.BlockSpec((BR, Dout), lambda i: (i, 0)),
            pl.BlockSpec((BR, Dout), lambda i: (i, 0)),
        ],
        out_shape=[
            jax.ShapeDtypeStruct((N, Dout), jnp.float32),
            jax.ShapeDtypeStruct((N, Dout), jnp.float32),
        ],
    )(x, Ws, Wn, b)


def _mm_mid(hs, aggp, degp3, Ws, Wn, b, Dout):
    def body(hs_ref, agg_ref, deg_ref, ws_ref, wn_ref, b_ref, hso_ref, hwo_ref):
        inv = 1.0 / jnp.maximum(deg_ref[0] + deg_ref[1], 1.0)
        h = jnp.maximum(hs_ref[...] + (agg_ref[0] + agg_ref[1]) * inv, 0.0)
        hso_ref[...] = jnp.dot(h, ws_ref[...],
                               preferred_element_type=jnp.float32) + b_ref[...]
        hwo_ref[...] = jnp.dot(h, wn_ref[...],
                               preferred_element_type=jnp.float32)
    Din = hs.shape[1]
    return pl.pallas_call(
        body,
        grid=(N // BR,),
        in_specs=[
            pl.BlockSpec((BR, Din), lambda i: (i, 0)),
            pl.BlockSpec((NC, BR, Din), lambda i: (0, i, 0)),
            pl.BlockSpec((NC, BR, 1), lambda i: (0, i, 0)),
            pl.BlockSpec((Din, Dout), lambda i: (0, 0)),
            pl.BlockSpec((Din, Dout), lambda i: (0, 0)),
            pl.BlockSpec((1, Dout), lambda i: (0, 0)),
        ],
        out_specs=[
            pl.BlockSpec((BR, Dout), lambda i: (i, 0)),
            pl.BlockSpec((BR, Dout), lambda i: (i, 0)),
        ],
        out_shape=[
            jax.ShapeDtypeStruct((N, Dout), jnp.float32),
            jax.ShapeDtypeStruct((N, Dout), jnp.float32),
        ],
    )(hs, aggp, degp3, Ws, Wn, b)


def _mm_last(hs, aggp, degp3):
    D = hs.shape[1]
    def body(hs_ref, agg_ref, deg_ref, o_ref):
        inv = 1.0 / jnp.maximum(deg_ref[0] + deg_ref[1], 1.0)
        o_ref[...] = hs_ref[...] + (agg_ref[0] + agg_ref[1]) * inv
    return pl.pallas_call(
        body,
        grid=(N // BR,),
        in_specs=[
            pl.BlockSpec((BR, D), lambda i: (i, 0)),
            pl.BlockSpec((NC, BR, D), lambda i: (0, i, 0)),
            pl.BlockSpec((NC, BR, 1), lambda i: (0, i, 0)),
        ],
        out_specs=pl.BlockSpec((BR, D), lambda i: (i, 0)),
        out_shape=jax.ShapeDtypeStruct((N, D), jnp.float32),
    )(hs, aggp, degp3)


def kernel(x, edge_index, W_self0, W_neigh0, b0, W_self1, W_neigh1, b1,
           W_self2, W_neigh2, b2):
    src = edge_index[0].astype(jnp.int32)
    dst = edge_index[1].astype(jnp.int32)
    E = src.shape[0]
    nchunk = NS * (CPW0 + CPW1)
    ep = nchunk * CH
    assert ep >= E
    pad_src = jnp.arange(ep - E, dtype=jnp.int32) % N
    pad_dst = N + jnp.arange(ep - E, dtype=jnp.int32) % (ACC_R - N)
    srcm = jnp.concatenate([src, pad_src]).reshape(nchunk, CH)
    dstm = jnp.concatenate([dst, pad_dst]).reshape(nchunk, CH)
    zrows = jnp.zeros((TILE_R, D_HID), jnp.float32)
    zdeg = jnp.zeros((TILE_R,), jnp.float32)
    ones = jnp.ones((CH,), jnp.float32)

    sc_agg_deg = _make_sc_agg(D_HID, True)
    sc_agg = _make_sc_agg(D_HID, False)

    b0r = b0.reshape(1, D_HID)
    b1r = b1.reshape(1, D_HID)
    Ws2p = jnp.pad(W_self2, ((0, 0), (0, D_OUT_PAD - D_OUT)))
    Wn2p = jnp.pad(W_neigh2, ((0, 0), (0, D_OUT_PAD - D_OUT)))
    b2r = jnp.pad(b2, (0, D_OUT_PAD - D_OUT)).reshape(1, D_OUT_PAD)

    hs0, hw0 = _mm_first(x, W_self0, W_neigh0, b0r, D_HID)
    aggp0, degp = sc_agg_deg(hw0, srcm, dstm, zrows, zdeg, ones)
    degp3 = degp.reshape(NC, ACC_R, 1)
    hs1, hw1 = _mm_mid(hs0, aggp0, degp3, W_self1, W_neigh1, b1r, D_HID)
    aggp1 = sc_agg(hw1, srcm, dstm, zrows, zdeg, ones)
    hs2, hw2 = _mm_mid(hs1, aggp1, degp3, Ws2p, Wn2p, b2r, D_OUT_PAD)
    aggp2 = sc_agg(hw2, srcm, dstm, zrows, zdeg, ones)
    out = _mm_last(hs2, aggp2, degp3)
    return out[:, :D_OUT]
